# baseline (speedup 1.0000x reference)
import functools

import jax
import jax.numpy as jnp
from jax import lax
from jax.experimental import pallas as pl
from jax.experimental.pallas import tpu as pltpu
from jax.experimental.pallas import tpu_sc as plsc

NC = 2
NS = 16
NW = NC * NS
L = 16


def _sc_body(bpw, d, ksplit, uidx_hbm, midx_hbm, utop_hbm, ubot_hbm, memb_hbm,
             ub_hbm, mb_hbm,
             part_out, bsum_out,
             uidx_v, midx_v, urows_v, urows2_v, mrows_v, ubv, mbv, bsv, accv,
             sem_rows, sem_ub, sem_mb):
    wid = lax.axis_index("s") * NC + lax.axis_index("c")
    base = wid * bpw
    pltpu.sync_copy(uidx_hbm.at[pl.ds(base, bpw)], uidx_v)
    pltpu.sync_copy(midx_hbm.at[pl.ds(base, bpw)], midx_v)

    cub = pltpu.async_copy(ub_hbm.at[uidx_v], ubv, sem_ub)
    cmb = pltpu.async_copy(mb_hbm.at[midx_v], mbv, sem_mb)

    copies = []
    for g in range(bpw // L):
        uvec = uidx_v[pl.ds(g * L, L)]
        mvec = midx_v[pl.ds(g * L, L)]
        for k in range(L):
            i = g * L + k
            vu = uvec[k]
            vm = mvec[k]
            vt = jnp.minimum(vu, ksplit - 1)
            vb = jnp.maximum(vu - ksplit, 0)
            copies.append(pltpu.async_copy(
                utop_hbm.at[vt], urows_v.at[i], sem_rows))
            copies.append(pltpu.async_copy(
                ubot_hbm.at[vb // 8, vb % 8], urows2_v.at[i], sem_rows))
            copies.append(pltpu.async_copy(
                memb_hbm.at[vm // 8, vm % 8], mrows_v.at[i], sem_rows))

    cub.wait()
    cmb.wait()
    for c in range(bpw // L):
        bsv[pl.ds(c * L, L)] = ubv[pl.ds(c * L, L)] + mbv[pl.ds(c * L, L)]
    pltpu.sync_copy(bsv, bsum_out.at[pl.ds(base, bpw)])

    for cp in copies:
        cp.wait()

    acc = jnp.zeros((L,), jnp.float32)
    for g in range(bpw // L):
        uvec = uidx_v[pl.ds(g * L, L)]
        for k in range(L):
            i = g * L + k
            wf = (uvec[k] < ksplit).astype(jnp.float32)
            wv = jnp.full((L,), wf)
            nv = jnp.full((L,), 1.0 - wf)
            for c in range(d // L):
                urow = (urows_v[i, pl.ds(c * L, L)] * wv
                        + urows2_v[i, pl.ds(c * L, L)] * nv)
                acc = acc + urow * mrows_v[i, pl.ds(c * L, L)]
    accv[...] = acc
    pltpu.sync_copy(accv, part_out.at[wid])


def _tc_body(part_ref, bsum_ref, out_ref):
    s = jnp.sum(part_ref[...])
    x = s + bsum_ref[...]
    out_ref[...] = 1.0 / (1.0 + jnp.exp(-x))


@jax.jit
def kernel(inputs, user_emb, movie_emb, user_b, movie_b):
    b = inputs.shape[0]
    v = user_emb.shape[0]
    d = user_emb.shape[1]
    bpw = b // NW
    u_idx = inputs[:, 0].astype(jnp.int32)
    m_idx = inputs[:, 1].astype(jnp.int32)
    ubf = user_b.reshape(-1)
    mbf = movie_b.reshape(-1)
    ksplit = 73728
    utop = user_emb[:ksplit]
    ubot3 = user_emb[ksplit:].reshape((v - ksplit) // 8, 8, d)
    memb3 = movie_emb.reshape(v // 8, 8, d)

    mesh = plsc.VectorSubcoreMesh(core_axis_name="c", subcore_axis_name="s")
    part, bsum = pl.kernel(
        functools.partial(_sc_body, bpw, d, ksplit),
        out_type=[
            jax.ShapeDtypeStruct((NW, L), jnp.float32),
            jax.ShapeDtypeStruct((b,), jnp.float32),
        ],
        mesh=mesh,
        compiler_params=pltpu.CompilerParams(use_tc_tiling_on_sc=True),
        scratch_types=[
            pltpu.VMEM((bpw,), jnp.int32),
            pltpu.VMEM((bpw,), jnp.int32),
            pltpu.VMEM((bpw, d), jnp.float32),
            pltpu.VMEM((bpw, d), jnp.float32),
            pltpu.VMEM((bpw, d), jnp.float32),
            pltpu.VMEM((bpw,), jnp.float32),
            pltpu.VMEM((bpw,), jnp.float32),
            pltpu.VMEM((bpw,), jnp.float32),
            pltpu.VMEM((L,), jnp.float32),
            pltpu.SemaphoreType.DMA,
            pltpu.SemaphoreType.DMA,
            pltpu.SemaphoreType.DMA,
        ],
    )(u_idx, m_idx, utop, ubot3, memb3, ubf, mbf)

    out = pl.pallas_call(
        _tc_body,
        out_shape=jax.ShapeDtypeStruct((NW, bpw), jnp.float32),
    )(part, bsum.reshape(NW, bpw))
    return out.reshape(b, 1)


# speedup vs baseline: 2.7202x; 2.7202x over previous
import functools

import jax
import jax.numpy as jnp
from jax import lax
from jax.experimental import pallas as pl
from jax.experimental.pallas import tpu as pltpu
from jax.experimental.pallas import tpu_sc as plsc

NC = 2
NS = 16
NW = NC * NS
L = 16


def _sc_body(bpw, d, uidx_hbm, midx_hbm, uemb_hbm, memb_hbm, ub_hbm, mb_hbm,
             part_out, bsum_out,
             uidx_v, midx_v, urows_v, mrows_v, ubv, mbv, bsv, accv,
             sem_rows, sem_ub, sem_mb):
    wid = lax.axis_index("s") * NC + lax.axis_index("c")
    base = wid * bpw
    pltpu.sync_copy(uidx_hbm.at[pl.ds(base, bpw)], uidx_v)
    pltpu.sync_copy(midx_hbm.at[pl.ds(base, bpw)], midx_v)

    cub = pltpu.async_copy(ub_hbm.at[uidx_v], ubv, sem_ub)
    cmb = pltpu.async_copy(mb_hbm.at[midx_v], mbv, sem_mb)

    copies = []
    for g in range(bpw // L):
        uvec = uidx_v[pl.ds(g * L, L)]
        mvec = midx_v[pl.ds(g * L, L)]
        for k in range(L):
            i = g * L + k
            vu = uvec[k]
            vm = mvec[k]
            copies.append(pltpu.async_copy(
                uemb_hbm.at[vu], urows_v.at[i], sem_rows))
            copies.append(pltpu.async_copy(
                memb_hbm.at[vm // 8, vm % 8], mrows_v.at[i], sem_rows))

    cub.wait()
    cmb.wait()
    for c in range(bpw // L):
        bsv[pl.ds(c * L, L)] = ubv[pl.ds(c * L, L)] + mbv[pl.ds(c * L, L)]
    pltpu.sync_copy(bsv, bsum_out.at[pl.ds(base, bpw)])

    for cp in copies:
        cp.wait()

    acc = jnp.zeros((L,), jnp.float32)

    def body(i, acc):
        for c in range(d // L):
            acc = acc + urows_v[i, pl.ds(c * L, L)] * mrows_v[i, pl.ds(c * L, L)]
        return acc

    acc = lax.fori_loop(0, bpw, body, acc)
    accv[...] = acc
    pltpu.sync_copy(accv, part_out.at[wid])


def _tc_body(part_ref, bsum_ref, out_ref):
    s = jnp.sum(part_ref[...])
    x = s + bsum_ref[...]
    out_ref[...] = 1.0 / (1.0 + jnp.exp(-x))


@jax.jit
def kernel(inputs, user_emb, movie_emb, user_b, movie_b):
    b = inputs.shape[0]
    v = user_emb.shape[0]
    d = user_emb.shape[1]
    bpw = b // NW
    u_idx = inputs[:, 0].astype(jnp.int32)
    m_idx = inputs[:, 1].astype(jnp.int32)
    ubf = user_b.reshape(-1)
    mbf = movie_b.reshape(-1)
    memb3 = movie_emb.reshape(v // 8, 8, d)

    mesh = plsc.VectorSubcoreMesh(core_axis_name="c", subcore_axis_name="s")
    part, bsum = pl.kernel(
        functools.partial(_sc_body, bpw, d),
        out_type=[
            jax.ShapeDtypeStruct((NW, L), jnp.float32),
            jax.ShapeDtypeStruct((b,), jnp.float32),
        ],
        mesh=mesh,
        compiler_params=pltpu.CompilerParams(use_tc_tiling_on_sc=True),
        scratch_types=[
            pltpu.VMEM((bpw,), jnp.int32),
            pltpu.VMEM((bpw,), jnp.int32),
            pltpu.VMEM((bpw, d), jnp.float32),
            pltpu.VMEM((bpw, d), jnp.float32),
            pltpu.VMEM((bpw,), jnp.float32),
            pltpu.VMEM((bpw,), jnp.float32),
            pltpu.VMEM((bpw,), jnp.float32),
            pltpu.VMEM((L,), jnp.float32),
            pltpu.SemaphoreType.DMA,
            pltpu.SemaphoreType.DMA,
            pltpu.SemaphoreType.DMA,
        ],
    )(u_idx, m_idx, user_emb, memb3, ubf, mbf)

    out = pl.pallas_call(
        _tc_body,
        out_shape=jax.ShapeDtypeStruct((NW, bpw), jnp.float32),
    )(part, bsum.reshape(NW, bpw))
    return out.reshape(b, 1)
